# baseline (speedup 1.0000x reference)
import functools

import numpy as np
import jax
import jax.numpy as jnp
from jax import lax
from jax.experimental import pallas as pl
from jax.experimental.pallas import tpu as pltpu
from jax.experimental.pallas import tpu_sc as plsc

N_ASU = 4
HMAX = 60
GRID = 2 * HMAX + 1
GRID_SIZE = GRID ** 3
WL_MIN = 0.3
WL_MAX = 1.5
MAX_MULT = 5

CHUNK = 2000
NWORKERS = 32

_GCD_PAD = np.zeros(448, np.int32)
_GCD_PAD[:441] = np.gcd.outer(np.arange(21), np.arange(21)).astype(np.int32).reshape(-1)


def _rsqrt(x):
    i = lax.bitcast_convert_type(x, jnp.int32)
    i = jnp.int32(0x5F3759DF) - (i >> 1)
    y = lax.bitcast_convert_type(i, jnp.float32)
    for _ in range(3):
        y = y * (jnp.float32(1.5) - jnp.float32(0.5) * x * y * y)
    return y


def _make_sc_call(n_total, row0, n_rows):
    assert n_rows % CHUNK == 0 and row0 % 8 == 0
    nblocks = n_rows // CHUNK
    steps = (nblocks + NWORKERS - 1) // NWORKERS
    vecs = CHUNK // 16

    mesh = plsc.VectorSubcoreMesh(
        core_axis_name="c", subcore_axis_name="s", num_cores=2, num_subcores=16)

    @functools.partial(
        pl.kernel,
        out_type=(
            jax.ShapeDtypeStruct((n_rows * 15,), jnp.int32),
            jax.ShapeDtypeStruct((n_rows * 5,), jnp.float32),
            jax.ShapeDtypeStruct((n_rows * 5,), jnp.float32),
            jax.ShapeDtypeStruct((n_rows * 5,), jnp.int32),
        ),
        mesh=mesh,
        compiler_params=pltpu.CompilerParams(needs_layout_passes=False),
        scratch_types=[
            pltpu.VMEM((CHUNK,), jnp.int32),
            pltpu.VMEM((CHUNK,), jnp.int32),
            pltpu.VMEM((CHUNK,), jnp.int32),
            pltpu.VMEM((CHUNK,), jnp.int32),
            pltpu.VMEM((CHUNK,), jnp.float32),
            pltpu.VMEM((15 * CHUNK,), jnp.int32),
            pltpu.VMEM((5 * CHUNK,), jnp.float32),
            pltpu.VMEM((5 * CHUNK,), jnp.float32),
            pltpu.VMEM((5 * CHUNK,), jnp.int32),
            pltpu.VMEM((448,), jnp.int32),
            pltpu.VMEM((16,), jnp.float32),
            pltpu.VMEM((16,), jnp.float32),
        ],
    )
    def sc_call(hkl_hbm, asu_hbm, wl_hbm, dmin_hbm, cell_hbm, gcd_hbm,
                hklo_hbm, wlo_hbm, do_hbm, reflo_hbm,
                h_v, k_v, l_v, asu_v, wl_v, hklo_v, wlo_v, do_v, reflo_v,
                gcd_v, dmin_v, rcp_v):
        cid = lax.axis_index("c")
        sid = lax.axis_index("s")
        wid = sid * 2 + cid

        pltpu.sync_copy(gcd_hbm, gcd_v)
        pltpu.sync_copy(dmin_hbm, dmin_v)
        pltpu.sync_copy(cell_hbm, rcp_v)
        rcp_v[...] = jnp.float32(1.0) / rcp_v[...]

        def vec_body(i, carry):
            r = i * 16
            h = h_v[pl.ds(r, 16)]
            k = k_v[pl.ds(r, 16)]
            l = l_v[pl.ds(r, 16)]
            asu = asu_v[pl.ds(r, 16)]
            wl = wl_v[pl.ds(r, 16)]

            asu3 = asu * 3
            rh = plsc.load_gather(rcp_v, [asu3])
            rk = plsc.load_gather(rcp_v, [asu3 + 1])
            rl = plsc.load_gather(rcp_v, [asu3 + 2])
            dmin_g = plsc.load_gather(dmin_v, [asu])

            nz = (h != 0) | (k != 0) | (l != 0)
            g1 = plsc.load_gather(gcd_v, [h * 21 + k])
            g = plsc.load_gather(gcd_v, [g1 * 21 + l])
            gs = jnp.maximum(g, 1)
            h0 = h // gs
            k0 = k // gs
            l0 = l // gs
            wl0 = wl * g.astype(jnp.float32)
            xh = h0.astype(jnp.float32) * rh
            xk = k0.astype(jnp.float32) * rk
            xl = l0.astype(jnp.float32) * rl
            s2 = (xh * xh + xk * xk) + xl * xl
            s2 = jnp.where(s2 > jnp.float32(0.0), s2, jnp.float32(1.0))
            d0 = _rsqrt(s2)
            t1 = (d0 / dmin_g).astype(jnp.int32)
            t2 = (wl0 / jnp.float32(WL_MIN)).astype(jnp.int32)
            t3 = (wl0 / jnp.float32(WL_MAX)).astype(jnp.int32)
            n_max = jnp.minimum(t1, t2)
            n_min = t3 + 1

            for j in range(MAX_MULT):
                n_j = n_min + j
                n_j = jnp.where(n_j > n_max, 0, n_j)
                hj = h0 * n_j
                kj = k0 * n_j
                lj = l0 * n_j
                inr = ((jnp.abs(hj) <= HMAX) & (jnp.abs(kj) <= HMAX)
                       & (jnp.abs(lj) <= HMAX)
                       & ((hj != 0) | (kj != 0) | (lj != 0)))
                yh = hj.astype(jnp.float32) * rh
                yk = kj.astype(jnp.float32) * rk
                yl = lj.astype(jnp.float32) * rl
                s2j = (yh * yh + yk * yk) + yl * yl
                s2j = jnp.where(s2j > jnp.float32(0.0), s2j, jnp.float32(1.0))
                dj = _rsqrt(s2j)
                pres = inr & (dj >= dmin_g)
                flat = ((hj + HMAX) * GRID + (kj + HMAX)) * GRID + (lj + HMAX)
                refl = jnp.where(pres, asu * GRID_SIZE + flat, -1)
                refl = jnp.where(nz, refl, 0)
                njf = n_j.astype(jnp.float32)
                ninv = jnp.where(
                    pres,
                    jnp.float32(1.0) / jnp.where(pres, njf, jnp.float32(1.0)),
                    jnp.float32(0.0))
                hklo_v[pl.ds((3 * j) * CHUNK + r, 16)] = jnp.where(pres, hj, 0)
                hklo_v[pl.ds((3 * j + 1) * CHUNK + r, 16)] = jnp.where(pres, kj, 0)
                hklo_v[pl.ds((3 * j + 2) * CHUNK + r, 16)] = jnp.where(pres, lj, 0)
                wlo_v[pl.ds(j * CHUNK + r, 16)] = wl0 * ninv
                do_v[pl.ds(j * CHUNK + r, 16)] = d0 * ninv
                reflo_v[pl.ds(j * CHUNK + r, 16)] = refl
            return carry

        def blk_body(t, carry):
            blk = wid + t * NWORKERS

            @pl.when(blk < nblocks)
            def _():
                base = blk * CHUNK
                src = row0 + base
                pltpu.sync_copy(hkl_hbm.at[pl.ds(src, CHUNK)], h_v)
                pltpu.sync_copy(hkl_hbm.at[pl.ds(n_total + src, CHUNK)], k_v)
                pltpu.sync_copy(hkl_hbm.at[pl.ds(2 * n_total + src, CHUNK)], l_v)
                pltpu.sync_copy(asu_hbm.at[pl.ds(src, CHUNK)], asu_v)
                pltpu.sync_copy(wl_hbm.at[pl.ds(src, CHUNK)], wl_v)
                lax.fori_loop(jnp.int32(0), jnp.int32(vecs), vec_body, 0)
                for p in range(15):
                    pltpu.sync_copy(
                        hklo_v.at[pl.ds(p * CHUNK, CHUNK)],
                        hklo_hbm.at[pl.ds(p * n_rows + base, CHUNK)])
                for j in range(5):
                    pltpu.sync_copy(
                        wlo_v.at[pl.ds(j * CHUNK, CHUNK)],
                        wlo_hbm.at[pl.ds(j * n_rows + base, CHUNK)])
                    pltpu.sync_copy(
                        do_v.at[pl.ds(j * CHUNK, CHUNK)],
                        do_hbm.at[pl.ds(j * n_rows + base, CHUNK)])
                    pltpu.sync_copy(
                        reflo_v.at[pl.ds(j * CHUNK, CHUNK)],
                        reflo_hbm.at[pl.ds(j * n_rows + base, CHUNK)])

            return carry

        lax.fori_loop(jnp.int32(0), jnp.int32(steps), blk_body, 0)

    return sc_call


def kernel(asu_id, hkl, wavelength, dmin, cell):
    n = asu_id.shape[0]
    asu32 = asu_id[:, 0].astype(jnp.int32)
    hkl32 = hkl.astype(jnp.int32).T.reshape(-1)
    wl = wavelength[:, 0].astype(jnp.float32)
    dmin_pad = jnp.concatenate(
        [dmin.astype(jnp.float32), jnp.ones((16 - N_ASU,), jnp.float32)])
    cell_pad = jnp.concatenate(
        [cell.astype(jnp.float32).reshape(-1), jnp.ones((4,), jnp.float32)])
    gcd_tab = jnp.asarray(_GCD_PAD)

    nsplit = 2
    part = n // nsplit
    parts = [
        _make_sc_call(n, row0 * part, part)(
            hkl32, asu32, wl, dmin_pad, cell_pad, gcd_tab)
        for row0 in range(nsplit)
    ]

    def plane(x, p):
        return x[p * part:(p + 1) * part][:, None, None]

    def column(idx, p):
        return jnp.concatenate(
            [plane(parts[h][idx], p) for h in range(nsplit)], axis=0)

    def assemble(idx):
        return jnp.concatenate(
            [column(idx, j) for j in range(MAX_MULT)], axis=1)

    hkl_out = jnp.concatenate(
        [jnp.concatenate([column(0, 3 * j + c) for c in range(3)], axis=2)
         for j in range(MAX_MULT)],
        axis=1).astype(jnp.int64)
    return hkl_out, assemble(1), assemble(2), assemble(3)


# speedup vs baseline: 1.1413x; 1.1413x over previous
import functools

import numpy as np
import jax
import jax.numpy as jnp
from jax import lax
from jax.experimental import pallas as pl
from jax.experimental.pallas import tpu as pltpu
from jax.experimental.pallas import tpu_sc as plsc

N_ASU = 4
HMAX = 60
GRID = 2 * HMAX + 1
GRID_SIZE = GRID ** 3
WL_MIN = 0.3
WL_MAX = 1.5
MAX_MULT = 5

CHUNK = 2000
NWORKERS = 32

_GCD_PAD = np.zeros(448, np.int32)
_GCD_PAD[:441] = np.gcd.outer(np.arange(21), np.arange(21)).astype(np.int32).reshape(-1)


def _rsqrt(x):
    i = lax.bitcast_convert_type(x, jnp.int32)
    i = jnp.int32(0x5F3759DF) - (i >> 1)
    y = lax.bitcast_convert_type(i, jnp.float32)
    for _ in range(3):
        y = y * (jnp.float32(1.5) - jnp.float32(0.5) * x * y * y)
    return y


def _make_sc_call(n_total, row0, n_rows):
    assert n_rows % CHUNK == 0 and row0 % 8 == 0
    nblocks = n_rows // CHUNK
    steps = (nblocks + NWORKERS - 1) // NWORKERS
    vecs = CHUNK // 16

    mesh = plsc.VectorSubcoreMesh(
        core_axis_name="c", subcore_axis_name="s", num_cores=2, num_subcores=16)

    @functools.partial(
        pl.kernel,
        out_type=(
            jax.ShapeDtypeStruct((n_rows * 15,), jnp.int32),
            jax.ShapeDtypeStruct((n_rows * 5,), jnp.float32),
            jax.ShapeDtypeStruct((n_rows * 5,), jnp.float32),
            jax.ShapeDtypeStruct((n_rows * 5,), jnp.int32),
        ),
        mesh=mesh,
        compiler_params=pltpu.CompilerParams(needs_layout_passes=False),
        scratch_types=[
            pltpu.VMEM((CHUNK,), jnp.int32),
            pltpu.VMEM((CHUNK,), jnp.int32),
            pltpu.VMEM((CHUNK,), jnp.int32),
            pltpu.VMEM((CHUNK,), jnp.int32),
            pltpu.VMEM((CHUNK,), jnp.float32),
            pltpu.VMEM((15 * CHUNK,), jnp.int32),
            pltpu.VMEM((5 * CHUNK,), jnp.float32),
            pltpu.VMEM((5 * CHUNK,), jnp.float32),
            pltpu.VMEM((5 * CHUNK,), jnp.int32),
            pltpu.VMEM((448,), jnp.int32),
            pltpu.VMEM((16,), jnp.float32),
            pltpu.VMEM((16,), jnp.float32),
            pltpu.SemaphoreType.DMA,
            pltpu.SemaphoreType.DMA,
        ],
    )
    def sc_call(hkl_hbm, asu_hbm, wl_hbm, dmin_hbm, cell_hbm, gcd_hbm,
                hklo_hbm, wlo_hbm, do_hbm, reflo_hbm,
                h_v, k_v, l_v, asu_v, wl_v, hklo_v, wlo_v, do_v, reflo_v,
                gcd_v, dmin_v, rcp_v, sem_in, sem_out):
        cid = lax.axis_index("c")
        sid = lax.axis_index("s")
        wid = sid * 2 + cid

        pltpu.sync_copy(gcd_hbm, gcd_v)
        pltpu.sync_copy(dmin_hbm, dmin_v)
        pltpu.sync_copy(cell_hbm, rcp_v)
        rcp_v[...] = jnp.float32(1.0) / rcp_v[...]

        def vec_body(i, carry):
            r = i * 16
            h = h_v[pl.ds(r, 16)]
            k = k_v[pl.ds(r, 16)]
            l = l_v[pl.ds(r, 16)]
            asu = asu_v[pl.ds(r, 16)]
            wl = wl_v[pl.ds(r, 16)]

            asu3 = asu * 3
            rh = plsc.load_gather(rcp_v, [asu3])
            rk = plsc.load_gather(rcp_v, [asu3 + 1])
            rl = plsc.load_gather(rcp_v, [asu3 + 2])
            dmin_g = plsc.load_gather(dmin_v, [asu])

            nz = (h != 0) | (k != 0) | (l != 0)
            g1 = plsc.load_gather(gcd_v, [h * 21 + k])
            g = plsc.load_gather(gcd_v, [g1 * 21 + l])
            gs = jnp.maximum(g, 1)
            h0 = h // gs
            k0 = k // gs
            l0 = l // gs
            wl0 = wl * g.astype(jnp.float32)
            xh = h0.astype(jnp.float32) * rh
            xk = k0.astype(jnp.float32) * rk
            xl = l0.astype(jnp.float32) * rl
            s2 = (xh * xh + xk * xk) + xl * xl
            s2 = jnp.where(s2 > jnp.float32(0.0), s2, jnp.float32(1.0))
            d0 = _rsqrt(s2)
            t1 = (d0 / dmin_g).astype(jnp.int32)
            t2 = (wl0 / jnp.float32(WL_MIN)).astype(jnp.int32)
            t3 = (wl0 / jnp.float32(WL_MAX)).astype(jnp.int32)
            n_max = jnp.minimum(t1, t2)
            n_min = t3 + 1

            for j in range(MAX_MULT):
                n_j = n_min + j
                n_j = jnp.where(n_j > n_max, 0, n_j)
                hj = h0 * n_j
                kj = k0 * n_j
                lj = l0 * n_j
                inr = ((jnp.abs(hj) <= HMAX) & (jnp.abs(kj) <= HMAX)
                       & (jnp.abs(lj) <= HMAX)
                       & ((hj != 0) | (kj != 0) | (lj != 0)))
                yh = hj.astype(jnp.float32) * rh
                yk = kj.astype(jnp.float32) * rk
                yl = lj.astype(jnp.float32) * rl
                s2j = (yh * yh + yk * yk) + yl * yl
                s2j = jnp.where(s2j > jnp.float32(0.0), s2j, jnp.float32(1.0))
                dj = _rsqrt(s2j)
                pres = inr & (dj >= dmin_g)
                flat = ((hj + HMAX) * GRID + (kj + HMAX)) * GRID + (lj + HMAX)
                refl = jnp.where(pres, asu * GRID_SIZE + flat, -1)
                refl = jnp.where(nz, refl, 0)
                njf = n_j.astype(jnp.float32)
                ninv = jnp.where(
                    pres,
                    jnp.float32(1.0) / jnp.where(pres, njf, jnp.float32(1.0)),
                    jnp.float32(0.0))
                hklo_v[pl.ds((3 * j) * CHUNK + r, 16)] = jnp.where(pres, hj, 0)
                hklo_v[pl.ds((3 * j + 1) * CHUNK + r, 16)] = jnp.where(pres, kj, 0)
                hklo_v[pl.ds((3 * j + 2) * CHUNK + r, 16)] = jnp.where(pres, lj, 0)
                wlo_v[pl.ds(j * CHUNK + r, 16)] = wl0 * ninv
                do_v[pl.ds(j * CHUNK + r, 16)] = d0 * ninv
                reflo_v[pl.ds(j * CHUNK + r, 16)] = refl
            return carry

        def blk_body(t, carry):
            blk = wid + t * NWORKERS

            @pl.when(blk < nblocks)
            def _():
                base = blk * CHUNK
                src = row0 + base
                ins = [
                    pltpu.async_copy(hkl_hbm.at[pl.ds(src, CHUNK)], h_v, sem_in),
                    pltpu.async_copy(
                        hkl_hbm.at[pl.ds(n_total + src, CHUNK)], k_v, sem_in),
                    pltpu.async_copy(
                        hkl_hbm.at[pl.ds(2 * n_total + src, CHUNK)], l_v, sem_in),
                    pltpu.async_copy(asu_hbm.at[pl.ds(src, CHUNK)], asu_v, sem_in),
                    pltpu.async_copy(wl_hbm.at[pl.ds(src, CHUNK)], wl_v, sem_in),
                ]
                for c in ins:
                    c.wait()
                lax.fori_loop(jnp.int32(0), jnp.int32(vecs), vec_body, 0)
                outs = []
                for p in range(15):
                    outs.append(pltpu.async_copy(
                        hklo_v.at[pl.ds(p * CHUNK, CHUNK)],
                        hklo_hbm.at[pl.ds(p * n_rows + base, CHUNK)], sem_out))
                for j in range(5):
                    outs.append(pltpu.async_copy(
                        wlo_v.at[pl.ds(j * CHUNK, CHUNK)],
                        wlo_hbm.at[pl.ds(j * n_rows + base, CHUNK)], sem_out))
                    outs.append(pltpu.async_copy(
                        do_v.at[pl.ds(j * CHUNK, CHUNK)],
                        do_hbm.at[pl.ds(j * n_rows + base, CHUNK)], sem_out))
                    outs.append(pltpu.async_copy(
                        reflo_v.at[pl.ds(j * CHUNK, CHUNK)],
                        reflo_hbm.at[pl.ds(j * n_rows + base, CHUNK)], sem_out))
                for c in outs:
                    c.wait()

            return carry

        lax.fori_loop(jnp.int32(0), jnp.int32(steps), blk_body, 0)

    return sc_call


def kernel(asu_id, hkl, wavelength, dmin, cell):
    n = asu_id.shape[0]
    asu32 = asu_id[:, 0].astype(jnp.int32)
    hkl32 = hkl.astype(jnp.int32).T.reshape(-1)
    wl = wavelength[:, 0].astype(jnp.float32)
    dmin_pad = jnp.concatenate(
        [dmin.astype(jnp.float32), jnp.ones((16 - N_ASU,), jnp.float32)])
    cell_pad = jnp.concatenate(
        [cell.astype(jnp.float32).reshape(-1), jnp.ones((4,), jnp.float32)])
    gcd_tab = jnp.asarray(_GCD_PAD)

    nsplit = 2
    part = n // nsplit
    parts = [
        _make_sc_call(n, row0 * part, part)(
            hkl32, asu32, wl, dmin_pad, cell_pad, gcd_tab)
        for row0 in range(nsplit)
    ]

    def plane(x, p):
        return x[p * part:(p + 1) * part][:, None, None]

    def assemble(idx):
        return jnp.concatenate(
            [jnp.concatenate([plane(parts[h][idx], j)
                              for j in range(MAX_MULT)], axis=1)
             for h in range(nsplit)], axis=0)

    hkl_out = jnp.concatenate(
        [jnp.concatenate(
            [jnp.concatenate([plane(hklp, 3 * j + c) for c in range(3)], axis=2)
             for j in range(MAX_MULT)], axis=1)
         for hklp in (parts[h][0] for h in range(nsplit))],
        axis=0).astype(jnp.int64)
    return hkl_out, assemble(1), assemble(2), assemble(3)


# speedup vs baseline: 1.1772x; 1.0315x over previous
import functools

import numpy as np
import jax
import jax.numpy as jnp
from jax import lax
from jax.experimental import pallas as pl
from jax.experimental.pallas import tpu as pltpu
from jax.experimental.pallas import tpu_sc as plsc

N_ASU = 4
HMAX = 60
GRID = 2 * HMAX + 1
GRID_SIZE = GRID ** 3
WL_MIN = 0.3
WL_MAX = 1.5
MAX_MULT = 5

CHUNK = 2000
NWORKERS = 32

_GCD_PAD = np.zeros(448, np.int32)
_GCD_PAD[:441] = np.gcd.outer(np.arange(21), np.arange(21)).astype(np.int32).reshape(-1)


def _rsqrt(x):
    i = lax.bitcast_convert_type(x, jnp.int32)
    i = jnp.int32(0x5F3759DF) - (i >> 1)
    y = lax.bitcast_convert_type(i, jnp.float32)
    for _ in range(3):
        y = y * (jnp.float32(1.5) - jnp.float32(0.5) * x * y * y)
    return y


def _make_sc_call(n_total, row0, n_rows):
    assert n_rows % CHUNK == 0 and row0 % 8 == 0
    nblocks = n_rows // CHUNK
    steps = (nblocks + NWORKERS - 1) // NWORKERS
    vecs = CHUNK // 16

    mesh = plsc.VectorSubcoreMesh(
        core_axis_name="c", subcore_axis_name="s", num_cores=2, num_subcores=16)

    @functools.partial(
        pl.kernel,
        out_type=(
            jax.ShapeDtypeStruct((n_rows * 15,), jnp.int32),
            jax.ShapeDtypeStruct((n_rows * 5,), jnp.float32),
            jax.ShapeDtypeStruct((n_rows * 5,), jnp.float32),
            jax.ShapeDtypeStruct((n_rows * 5,), jnp.int32),
        ),
        mesh=mesh,
        compiler_params=pltpu.CompilerParams(needs_layout_passes=False),
        scratch_types=[
            pltpu.VMEM((CHUNK,), jnp.int32),
            pltpu.VMEM((CHUNK,), jnp.int32),
            pltpu.VMEM((CHUNK,), jnp.int32),
            pltpu.VMEM((CHUNK,), jnp.int32),
            pltpu.VMEM((CHUNK,), jnp.float32),
            pltpu.VMEM((15 * CHUNK,), jnp.int32),
            pltpu.VMEM((5 * CHUNK,), jnp.float32),
            pltpu.VMEM((5 * CHUNK,), jnp.float32),
            pltpu.VMEM((5 * CHUNK,), jnp.int32),
            pltpu.VMEM((448,), jnp.int32),
            pltpu.VMEM((16,), jnp.float32),
            pltpu.VMEM((16,), jnp.float32),
            pltpu.SemaphoreType.DMA,
            pltpu.SemaphoreType.DMA,
        ],
    )
    def sc_call(hkl_hbm, asu_hbm, wl_hbm, dmin_hbm, cell_hbm, gcd_hbm,
                hklo_hbm, wlo_hbm, do_hbm, reflo_hbm,
                h_v, k_v, l_v, asu_v, wl_v, hklo_v, wlo_v, do_v, reflo_v,
                gcd_v, dmin_v, rcp_v, sem_in, sem_out):
        cid = lax.axis_index("c")
        sid = lax.axis_index("s")
        wid = sid * 2 + cid

        pltpu.sync_copy(gcd_hbm, gcd_v)
        pltpu.sync_copy(dmin_hbm, dmin_v)
        pltpu.sync_copy(cell_hbm, rcp_v)
        rcp_v[...] = jnp.float32(1.0) / rcp_v[...]

        def vec_body(i):
            r = i * 16
            h = h_v[pl.ds(r, 16)]
            k = k_v[pl.ds(r, 16)]
            l = l_v[pl.ds(r, 16)]
            asu = asu_v[pl.ds(r, 16)]
            wl = wl_v[pl.ds(r, 16)]

            asu3 = asu * 3
            rh = plsc.load_gather(rcp_v, [asu3])
            rk = plsc.load_gather(rcp_v, [asu3 + 1])
            rl = plsc.load_gather(rcp_v, [asu3 + 2])
            dmin_g = plsc.load_gather(dmin_v, [asu])

            nz = (h != 0) | (k != 0) | (l != 0)
            g1 = plsc.load_gather(gcd_v, [h * 21 + k])
            g = plsc.load_gather(gcd_v, [g1 * 21 + l])
            gs = jnp.maximum(g, 1)
            h0 = h // gs
            k0 = k // gs
            l0 = l // gs
            wl0 = wl * g.astype(jnp.float32)
            xh = h0.astype(jnp.float32) * rh
            xk = k0.astype(jnp.float32) * rk
            xl = l0.astype(jnp.float32) * rl
            s2 = (xh * xh + xk * xk) + xl * xl
            s2 = jnp.where(s2 > jnp.float32(0.0), s2, jnp.float32(1.0))
            d0 = _rsqrt(s2)
            t1 = (d0 / dmin_g).astype(jnp.int32)
            t2 = (wl0 / jnp.float32(WL_MIN)).astype(jnp.int32)
            t3 = (wl0 / jnp.float32(WL_MAX)).astype(jnp.int32)
            n_max = jnp.minimum(t1, t2)
            n_min = t3 + 1

            for j in range(MAX_MULT):
                n_j = n_min + j
                n_j = jnp.where(n_j > n_max, 0, n_j)
                hj = h0 * n_j
                kj = k0 * n_j
                lj = l0 * n_j
                inr = ((jnp.abs(hj) <= HMAX) & (jnp.abs(kj) <= HMAX)
                       & (jnp.abs(lj) <= HMAX)
                       & ((hj != 0) | (kj != 0) | (lj != 0)))
                yh = hj.astype(jnp.float32) * rh
                yk = kj.astype(jnp.float32) * rk
                yl = lj.astype(jnp.float32) * rl
                s2j = (yh * yh + yk * yk) + yl * yl
                s2j = jnp.where(s2j > jnp.float32(0.0), s2j, jnp.float32(1.0))
                dj = _rsqrt(s2j)
                pres = inr & (dj >= dmin_g)
                flat = ((hj + HMAX) * GRID + (kj + HMAX)) * GRID + (lj + HMAX)
                refl = jnp.where(pres, asu * GRID_SIZE + flat, -1)
                refl = jnp.where(nz, refl, 0)
                njf = n_j.astype(jnp.float32)
                ninv = jnp.where(
                    pres,
                    jnp.float32(1.0) / jnp.where(pres, njf, jnp.float32(1.0)),
                    jnp.float32(0.0))
                hklo_v[pl.ds((3 * j) * CHUNK + r, 16)] = jnp.where(pres, hj, 0)
                hklo_v[pl.ds((3 * j + 1) * CHUNK + r, 16)] = jnp.where(pres, kj, 0)
                hklo_v[pl.ds((3 * j + 2) * CHUNK + r, 16)] = jnp.where(pres, lj, 0)
                wlo_v[pl.ds(j * CHUNK + r, 16)] = wl0 * ninv
                do_v[pl.ds(j * CHUNK + r, 16)] = d0 * ninv
                reflo_v[pl.ds(j * CHUNK + r, 16)] = refl

        def blk_body(t, carry):
            blk = wid + t * NWORKERS

            @pl.when(blk < nblocks)
            def _():
                base = blk * CHUNK
                src = row0 + base
                ins = [
                    pltpu.async_copy(hkl_hbm.at[pl.ds(src, CHUNK)], h_v, sem_in),
                    pltpu.async_copy(
                        hkl_hbm.at[pl.ds(n_total + src, CHUNK)], k_v, sem_in),
                    pltpu.async_copy(
                        hkl_hbm.at[pl.ds(2 * n_total + src, CHUNK)], l_v, sem_in),
                    pltpu.async_copy(asu_hbm.at[pl.ds(src, CHUNK)], asu_v, sem_in),
                    pltpu.async_copy(wl_hbm.at[pl.ds(src, CHUNK)], wl_v, sem_in),
                ]
                for c in ins:
                    c.wait()
                plsc.parallel_loop(
                    jnp.int32(0), jnp.int32(vecs), step=jnp.int32(1),
                    unroll=2)(vec_body)
                outs = []
                for p in range(15):
                    outs.append(pltpu.async_copy(
                        hklo_v.at[pl.ds(p * CHUNK, CHUNK)],
                        hklo_hbm.at[pl.ds(p * n_rows + base, CHUNK)], sem_out))
                for j in range(5):
                    outs.append(pltpu.async_copy(
                        wlo_v.at[pl.ds(j * CHUNK, CHUNK)],
                        wlo_hbm.at[pl.ds(j * n_rows + base, CHUNK)], sem_out))
                    outs.append(pltpu.async_copy(
                        do_v.at[pl.ds(j * CHUNK, CHUNK)],
                        do_hbm.at[pl.ds(j * n_rows + base, CHUNK)], sem_out))
                    outs.append(pltpu.async_copy(
                        reflo_v.at[pl.ds(j * CHUNK, CHUNK)],
                        reflo_hbm.at[pl.ds(j * n_rows + base, CHUNK)], sem_out))
                for c in outs:
                    c.wait()

            return carry

        lax.fori_loop(jnp.int32(0), jnp.int32(steps), blk_body, 0)

    return sc_call


def kernel(asu_id, hkl, wavelength, dmin, cell):
    n = asu_id.shape[0]
    asu32 = asu_id[:, 0].astype(jnp.int32)
    hkl32 = hkl.astype(jnp.int32).T.reshape(-1)
    wl = wavelength[:, 0].astype(jnp.float32)
    dmin_pad = jnp.concatenate(
        [dmin.astype(jnp.float32), jnp.ones((16 - N_ASU,), jnp.float32)])
    cell_pad = jnp.concatenate(
        [cell.astype(jnp.float32).reshape(-1), jnp.ones((4,), jnp.float32)])
    gcd_tab = jnp.asarray(_GCD_PAD)

    nsplit = 2
    part = n // nsplit
    parts = [
        _make_sc_call(n, row0 * part, part)(
            hkl32, asu32, wl, dmin_pad, cell_pad, gcd_tab)
        for row0 in range(nsplit)
    ]

    def plane(x, p):
        return x[p * part:(p + 1) * part][:, None, None]

    def assemble(idx):
        return jnp.concatenate(
            [jnp.concatenate([plane(parts[h][idx], j)
                              for j in range(MAX_MULT)], axis=1)
             for h in range(nsplit)], axis=0)

    hkl_out = jnp.concatenate(
        [jnp.concatenate(
            [jnp.concatenate([plane(hklp, 3 * j + c) for c in range(3)], axis=2)
             for j in range(MAX_MULT)], axis=1)
         for hklp in (parts[h][0] for h in range(nsplit))],
        axis=0).astype(jnp.int64)
    return hkl_out, assemble(1), assemble(2), assemble(3)
